# baseline (speedup 1.0000x reference)
import functools

import jax
import jax.numpy as jnp
from jax import lax
from jax.experimental import pallas as pl
from jax.experimental.pallas import tpu as pltpu
from jax.experimental.pallas import tpu_sc as plsc

N = 10000
E = 320000
D = 128
H = 64
G = 16
OUT = 64

NC = 2
NS = 16
NW = NC * NS
L = 16

K = 128
CH = 80
EW = K * CH
EP = EW * NW
NP = 10240
RPT = NP // NS
R1 = 2048
GP = NP // R1

_mesh = plsc.VectorSubcoreMesh(
    core_axis_name="c", subcore_axis_name="s", num_cores=NC, num_subcores=NS)

_sc_params = pltpu.CompilerParams(use_tc_tiling_on_sc=False)

_zeros16 = functools.partial(jnp.zeros, (L,), jnp.float32)


def _deg_body(dst_hbm, out_hbm, dst_v, ones_v, zbuf_v, deg_sh):
    cid = lax.axis_index("c")
    sid = lax.axis_index("s")
    wid = sid * NC + cid

    for t in range(K // L):
        ones_v[pl.ds(t * L, L)] = jnp.ones((L,), jnp.float32)

    def zstore(i, _):
        zbuf_v[pl.ds(i * L, L)] = _zeros16()
        return 0
    lax.fori_loop(0, RPT // L, zstore, 0)

    base = sid * RPT
    pltpu.sync_copy(zbuf_v, deg_sh.at[pl.ds(base, RPT)])
    pltpu.sync_copy(dst_hbm.at[wid], dst_v)
    plsc.subcore_barrier()

    def chunk(j, _):
        pltpu.sync_copy(ones_v, deg_sh.at[dst_v.at[j]], add=True)
        return 0
    lax.fori_loop(0, CH, chunk, 0)

    plsc.subcore_barrier()
    pltpu.sync_copy(deg_sh.at[pl.ds(base, RPT)],
                    out_hbm.at[cid, pl.ds(base, RPT)])


_deg_call = pl.kernel(
    _deg_body,
    out_type=jax.ShapeDtypeStruct((NC, NP), jnp.float32),
    mesh=_mesh,
    scratch_types=[
        pltpu.VMEM((CH, K), jnp.int32),
        pltpu.VMEM((K,), jnp.float32),
        pltpu.VMEM((RPT,), jnp.float32),
        pltpu.VMEM_SHARED((NP,), jnp.float32),
    ],
    compiler_params=_sc_params,
)


def _agg_body(src_hbm, dst_hbm, y_hbm, out_hbm, src_v, dst_v, rows_v, agg_sh,
              gs0, gs1, gs2, gs3, ss0, ss1, ss2, ss3):
    gsem = [gs0, gs1, gs2, gs3]
    ssem = [ss0, ss1, ss2, ss3]
    cid = lax.axis_index("c")
    sid = lax.axis_index("s")
    wid = sid * NC + cid

    def zstore(r, _):
        for c in range(H // (2 * L)):
            rows_v[0, r, pl.ds(c * 2 * L, 2 * L)] = jnp.zeros(
                (2 * L,), jnp.bfloat16)
        return 0
    lax.fori_loop(0, K, zstore, 0)

    base = sid * RPT
    for t in range(RPT // K):
        pltpu.sync_copy(rows_v.at[0], agg_sh.at[pl.ds(base + t * K, K)])
    pltpu.sync_copy(src_hbm.at[wid], src_v)
    pltpu.sync_copy(dst_hbm.at[wid], dst_v)
    plsc.subcore_barrier()

    def gather(j, buf):
        pltpu.async_copy(y_hbm.at[src_v.at[j]], rows_v.at[buf], gsem[buf])

    def gwait(buf):
        pltpu.make_async_copy(y_hbm.at[src_v.at[0]], rows_v.at[buf],
                              gsem[buf]).wait()

    def scat(j, buf):
        pltpu.async_copy(rows_v.at[buf], agg_sh.at[dst_v.at[j]], ssem[buf],
                         add=True)

    def swait(buf):
        pltpu.make_async_copy(rows_v.at[buf], agg_sh.at[pl.ds(0, K)],
                              ssem[buf]).wait()

    NB = 4
    gather(0, 0)
    gather(1, 1)

    def chunk4(jj, _):
        j = jj * NB
        for u in range(NB):
            b = u
            gwait(b)
            scat(j + u, b)
            bn = (u + 2) % NB
            if u >= 2:
                swait(bn)
            else:
                @pl.when(jj > 0)
                def _():
                    swait(bn)
            gather((j + u + 2) % CH, bn)
        return 0
    lax.fori_loop(0, CH // NB, chunk4, 0)

    gwait(0)
    gwait(1)
    swait(2)
    swait(3)

    plsc.subcore_barrier()
    pltpu.sync_copy(agg_sh.at[pl.ds(base, RPT)],
                    out_hbm.at[cid, pl.ds(base, RPT)])


_agg_call = pl.kernel(
    _agg_body,
    out_type=jax.ShapeDtypeStruct((NC, NP, H), jnp.bfloat16),
    mesh=_mesh,
    scratch_types=[
        pltpu.VMEM((CH, K), jnp.int32),
        pltpu.VMEM((CH, K), jnp.int32),
        pltpu.VMEM((4, K, H), jnp.bfloat16),
        pltpu.VMEM_SHARED((NP, H), jnp.bfloat16),
    ] + [pltpu.SemaphoreType.DMA] * 8,
    compiler_params=_sc_params,
)


RW = R1 // 128

_deg_spec = pl.BlockSpec((NC, RW, 128), lambda i: (0, i, 0))


def _col(mat):
    ident = (lax.broadcasted_iota(jnp.int32, (128, 128), 0) ==
             lax.broadcasted_iota(jnp.int32, (128, 128), 1)
             ).astype(jnp.float32)
    mt = lax.dot_general(ident, mat, (((1,), (1,)), ((), ())),
                         preferred_element_type=jnp.float32)
    return jnp.concatenate([mt[:, a:a + 1] for a in range(RW)], axis=0)


def _dinv_col(deg_ref):
    return _col(lax.rsqrt(deg_ref[0] + deg_ref[1] + 1.0))


def _mm_body(x_ref, w_ref, out_ref):
    out_ref[...] = jnp.dot(x_ref[...], w_ref[...],
                           preferred_element_type=jnp.float32)


def _mm(xp, W1):
    return pl.pallas_call(
        _mm_body,
        grid=(GP,),
        in_specs=[
            pl.BlockSpec((R1, D), lambda i: (i, 0)),
            pl.BlockSpec((D, H), lambda i: (0, 0)),
        ],
        out_specs=pl.BlockSpec((R1, H), lambda i: (i, 0)),
        out_shape=jax.ShapeDtypeStruct((NP, H), jnp.float32),
    )(xp, W1)


def _scale_body(xw_ref, deg_ref, out_ref, outh_ref):
    dinv = _dinv_col(deg_ref)
    y = xw_ref[...] * dinv
    out_ref[...] = y
    outh_ref[...] = y.astype(jnp.bfloat16)


def _scale(xw, deg2d):
    return pl.pallas_call(
        _scale_body,
        grid=(GP,),
        in_specs=[
            pl.BlockSpec((R1, H), lambda i: (i, 0)),
            _deg_spec,
        ],
        out_specs=[
            pl.BlockSpec((R1, H), lambda i: (i, 0)),
            pl.BlockSpec((R1, H), lambda i: (i, 0)),
        ],
        out_shape=[
            jax.ShapeDtypeStruct((NP, H), jnp.float32),
            jax.ShapeDtypeStruct((NP, H), jnp.bfloat16),
        ],
    )(xw, deg2d)


def _layer_body(agg_ref, y_ref, deg_ref, b_ref, w_ref, out_ref, outh_ref):
    dinv = _dinv_col(deg_ref)
    agg = agg_ref[0].astype(jnp.float32) + agg_ref[1].astype(jnp.float32)
    h = jnp.maximum((agg + y_ref[...]) * dinv + b_ref[...], 0.0)
    y2 = jnp.dot(h, w_ref[...], preferred_element_type=jnp.float32) * dinv
    out_ref[...] = y2
    outh_ref[...] = y2.astype(jnp.bfloat16)


def _layer(agg, y, deg2d, b_row, W2):
    return pl.pallas_call(
        _layer_body,
        grid=(GP,),
        in_specs=[
            pl.BlockSpec((NC, R1, H), lambda i: (0, i, 0)),
            pl.BlockSpec((R1, H), lambda i: (i, 0)),
            _deg_spec,
            pl.BlockSpec((1, H), lambda i: (0, 0)),
            pl.BlockSpec((H, H), lambda i: (0, 0)),
        ],
        out_specs=[
            pl.BlockSpec((R1, H), lambda i: (i, 0)),
            pl.BlockSpec((R1, H), lambda i: (i, 0)),
        ],
        out_shape=[
            jax.ShapeDtypeStruct((NP, H), jnp.float32),
            jax.ShapeDtypeStruct((NP, H), jnp.bfloat16),
        ],
    )(agg, y, deg2d, b_row, W2)


def _final_body(agg_ref, y_ref, deg_ref, b_ref, batch_ref, wo_ref, bo_ref,
                out_ref, sums_ref, cnts_ref):
    i = pl.program_id(0)

    @pl.when(i == 0)
    def _():
        sums_ref[...] = jnp.zeros_like(sums_ref)
        cnts_ref[...] = jnp.zeros_like(cnts_ref)

    dinv = _dinv_col(deg_ref)
    agg = agg_ref[0].astype(jnp.float32) + agg_ref[1].astype(jnp.float32)
    h = jnp.maximum((agg + y_ref[...]) * dinv + b_ref[...], 0.0)
    seg = _col(batch_ref[...].astype(jnp.float32))
    gids = lax.broadcasted_iota(jnp.int32, (1, G), 1).astype(jnp.float32)
    oh = (seg == gids).astype(jnp.float32)
    sums_ref[...] += lax.dot_general(
        oh, h, (((0,), (0,)), ((), ())), preferred_element_type=jnp.float32)
    cnts_ref[...] += lax.dot_general(
        oh, jnp.ones((R1, 1), jnp.float32), (((0,), (0,)), ((), ())),
        preferred_element_type=jnp.float32)

    @pl.when(i == GP - 1)
    def _():
        pooled = sums_ref[...] / jnp.maximum(cnts_ref[...], 1.0)
        out_ref[...] = jnp.dot(
            pooled, wo_ref[...], preferred_element_type=jnp.float32
        ) + bo_ref[...]


def _final(agg, y, deg2d, b_row, batch2d, Wo, bo_row):
    return pl.pallas_call(
        _final_body,
        grid=(GP,),
        in_specs=[
            pl.BlockSpec((NC, R1, H), lambda i: (0, i, 0)),
            pl.BlockSpec((R1, H), lambda i: (i, 0)),
            _deg_spec,
            pl.BlockSpec((1, H), lambda i: (0, 0)),
            pl.BlockSpec((RW, 128), lambda i: (i, 0)),
            pl.BlockSpec((H, OUT), lambda i: (0, 0)),
            pl.BlockSpec((1, OUT), lambda i: (0, 0)),
        ],
        out_specs=pl.BlockSpec((G, OUT), lambda i: (0, 0)),
        out_shape=jax.ShapeDtypeStruct((G, OUT), jnp.float32),
        scratch_shapes=[
            pltpu.VMEM((G, H), jnp.float32),
            pltpu.VMEM((G, 1), jnp.float32),
        ],
    )(agg, y, deg2d, b_row, batch2d, Wo, bo_row)


def kernel(x, edge_index, batch, W1, b1, W2, b2, Wo, bo):
    xp = jnp.pad(x, ((0, NP - N), (0, 0)))
    padrows = (N + (jnp.arange(EP - E, dtype=jnp.int32) % (NP - N))
               ).astype(jnp.int32)
    srcp = jnp.concatenate([edge_index[0], padrows]).reshape(NW, CH, K)
    dstp = jnp.concatenate([edge_index[1], padrows]).reshape(NW, CH, K)
    batch2d = jnp.concatenate(
        [batch, jnp.full((NP - N,), G, jnp.int32)]).reshape(NP // 128, 128)

    xw1 = _mm(xp, W1)
    deg_part = _deg_call(dstp)
    deg2d = deg_part.reshape(NC, NP // 128, 128)
    y1, y1h = _scale(xw1, deg2d)
    agg1 = _agg_call(srcp, dstp, y1h)
    y2, y2h = _layer(agg1, y1, deg2d, b1.reshape(1, H), W2)
    agg2 = _agg_call(srcp, dstp, y2h)
    return _final(agg2, y2, deg2d, b2.reshape(1, H), batch2d,
                  Wo, bo.reshape(1, OUT))


# speedup vs baseline: 1.0850x; 1.0850x over previous
import functools

import jax
import jax.numpy as jnp
from jax import lax
from jax.experimental import pallas as pl
from jax.experimental.pallas import tpu as pltpu
from jax.experimental.pallas import tpu_sc as plsc

N = 10000
E = 320000
D = 128
H = 64
G = 16
OUT = 64

NC = 2
NS = 16
NW = NC * NS
L = 16

K = 256
CH = 40
EW = K * CH
EP = EW * NW
NP = 10240
RPT = NP // NS
R1 = 2048
GP = NP // R1

_mesh = plsc.VectorSubcoreMesh(
    core_axis_name="c", subcore_axis_name="s", num_cores=NC, num_subcores=NS)

_sc_params = pltpu.CompilerParams(use_tc_tiling_on_sc=False)

_zeros16 = functools.partial(jnp.zeros, (L,), jnp.float32)


def _deg_body(dst_hbm, out_hbm, dst_v, ones_v, zbuf_v, deg_sh):
    cid = lax.axis_index("c")
    sid = lax.axis_index("s")
    wid = sid * NC + cid

    for t in range(K // L):
        ones_v[pl.ds(t * L, L)] = jnp.ones((L,), jnp.float32)

    def zstore(i, _):
        zbuf_v[pl.ds(i * L, L)] = _zeros16()
        return 0
    lax.fori_loop(0, RPT // L, zstore, 0)

    base = sid * RPT
    pltpu.sync_copy(zbuf_v, deg_sh.at[pl.ds(base, RPT)])
    pltpu.sync_copy(dst_hbm.at[wid], dst_v)
    plsc.subcore_barrier()

    def chunk(j, _):
        pltpu.sync_copy(ones_v, deg_sh.at[dst_v.at[j]], add=True)
        return 0
    lax.fori_loop(0, CH, chunk, 0)

    plsc.subcore_barrier()
    pltpu.sync_copy(deg_sh.at[pl.ds(base, RPT)],
                    out_hbm.at[cid, pl.ds(base, RPT)])


_deg_call = pl.kernel(
    _deg_body,
    out_type=jax.ShapeDtypeStruct((NC, NP), jnp.float32),
    mesh=_mesh,
    scratch_types=[
        pltpu.VMEM((CH, K), jnp.int32),
        pltpu.VMEM((K,), jnp.float32),
        pltpu.VMEM((RPT,), jnp.float32),
        pltpu.VMEM_SHARED((NP,), jnp.float32),
    ],
    compiler_params=_sc_params,
)


def _agg_body(src_hbm, dst_hbm, y_hbm, out_hbm, src_v, dst_v, rows_v, agg_sh,
              gs0, gs1, gs2, gs3, ss0, ss1, ss2, ss3):
    gsem = [gs0, gs1, gs2, gs3]
    ssem = [ss0, ss1, ss2, ss3]
    cid = lax.axis_index("c")
    sid = lax.axis_index("s")
    wid = sid * NC + cid

    def zstore(r, _):
        for c in range(H // (2 * L)):
            rows_v[0, r, pl.ds(c * 2 * L, 2 * L)] = jnp.zeros(
                (2 * L,), jnp.bfloat16)
        return 0
    lax.fori_loop(0, K, zstore, 0)

    base = sid * RPT
    for t in range(RPT // 128):
        pltpu.sync_copy(rows_v.at[0, pl.ds(0, 128)],
                        agg_sh.at[pl.ds(base + t * 128, 128)])
    pltpu.sync_copy(src_hbm.at[wid], src_v)
    pltpu.sync_copy(dst_hbm.at[wid], dst_v)
    plsc.subcore_barrier()

    def gather(j, buf):
        pltpu.async_copy(y_hbm.at[src_v.at[j]], rows_v.at[buf], gsem[buf])

    def gwait(buf):
        pltpu.make_async_copy(y_hbm.at[src_v.at[0]], rows_v.at[buf],
                              gsem[buf]).wait()

    def scat(j, buf):
        pltpu.async_copy(rows_v.at[buf], agg_sh.at[dst_v.at[j]], ssem[buf],
                         add=True)

    def swait(buf):
        pltpu.make_async_copy(rows_v.at[buf], agg_sh.at[pl.ds(0, K)],
                              ssem[buf]).wait()

    NB = 4
    gather(0, 0)
    gather(1, 1)

    def chunk4(jj, _):
        j = jj * NB
        for u in range(NB):
            b = u
            gwait(b)
            scat(j + u, b)
            bn = (u + 2) % NB
            if u >= 2:
                swait(bn)
            else:
                @pl.when(jj > 0)
                def _():
                    swait(bn)
            gather((j + u + 2) % CH, bn)
        return 0
    lax.fori_loop(0, CH // NB, chunk4, 0)

    gwait(0)
    gwait(1)
    swait(2)
    swait(3)

    plsc.subcore_barrier()
    pltpu.sync_copy(agg_sh.at[pl.ds(base, RPT)],
                    out_hbm.at[cid, pl.ds(base, RPT)])


_agg_call = pl.kernel(
    _agg_body,
    out_type=jax.ShapeDtypeStruct((NC, NP, H), jnp.bfloat16),
    mesh=_mesh,
    scratch_types=[
        pltpu.VMEM((CH, K), jnp.int32),
        pltpu.VMEM((CH, K), jnp.int32),
        pltpu.VMEM((4, K, H), jnp.bfloat16),
        pltpu.VMEM_SHARED((NP, H), jnp.bfloat16),
    ] + [pltpu.SemaphoreType.DMA] * 8,
    compiler_params=_sc_params,
)


RW = R1 // 128

_deg_spec = pl.BlockSpec((NC, RW, 128), lambda i: (0, i, 0))


def _col(mat):
    ident = (lax.broadcasted_iota(jnp.int32, (128, 128), 0) ==
             lax.broadcasted_iota(jnp.int32, (128, 128), 1)
             ).astype(jnp.float32)
    mt = lax.dot_general(ident, mat, (((1,), (1,)), ((), ())),
                         preferred_element_type=jnp.float32)
    return jnp.concatenate([mt[:, a:a + 1] for a in range(RW)], axis=0)


def _dinv_col(deg_ref):
    return _col(lax.rsqrt(deg_ref[0] + deg_ref[1] + 1.0))


def _mm_body(x_ref, w_ref, out_ref):
    out_ref[...] = jnp.dot(x_ref[...], w_ref[...],
                           preferred_element_type=jnp.float32)


def _mm(xp, W1):
    return pl.pallas_call(
        _mm_body,
        grid=(GP,),
        in_specs=[
            pl.BlockSpec((R1, D), lambda i: (i, 0)),
            pl.BlockSpec((D, H), lambda i: (0, 0)),
        ],
        out_specs=pl.BlockSpec((R1, H), lambda i: (i, 0)),
        out_shape=jax.ShapeDtypeStruct((NP, H), jnp.float32),
    )(xp, W1)


def _scale_body(xw_ref, deg_ref, out_ref, outh_ref):
    dinv = _dinv_col(deg_ref)
    y = xw_ref[...] * dinv
    out_ref[...] = y
    outh_ref[...] = y.astype(jnp.bfloat16)


def _scale(xw, deg2d):
    return pl.pallas_call(
        _scale_body,
        grid=(GP,),
        in_specs=[
            pl.BlockSpec((R1, H), lambda i: (i, 0)),
            _deg_spec,
        ],
        out_specs=[
            pl.BlockSpec((R1, H), lambda i: (i, 0)),
            pl.BlockSpec((R1, H), lambda i: (i, 0)),
        ],
        out_shape=[
            jax.ShapeDtypeStruct((NP, H), jnp.float32),
            jax.ShapeDtypeStruct((NP, H), jnp.bfloat16),
        ],
    )(xw, deg2d)


def _layer_body(agg_ref, y_ref, deg_ref, b_ref, w_ref, out_ref, outh_ref):
    dinv = _dinv_col(deg_ref)
    agg = agg_ref[0].astype(jnp.float32) + agg_ref[1].astype(jnp.float32)
    h = jnp.maximum((agg + y_ref[...]) * dinv + b_ref[...], 0.0)
    y2 = jnp.dot(h, w_ref[...], preferred_element_type=jnp.float32) * dinv
    out_ref[...] = y2
    outh_ref[...] = y2.astype(jnp.bfloat16)


def _layer(agg, y, deg2d, b_row, W2):
    return pl.pallas_call(
        _layer_body,
        grid=(GP,),
        in_specs=[
            pl.BlockSpec((NC, R1, H), lambda i: (0, i, 0)),
            pl.BlockSpec((R1, H), lambda i: (i, 0)),
            _deg_spec,
            pl.BlockSpec((1, H), lambda i: (0, 0)),
            pl.BlockSpec((H, H), lambda i: (0, 0)),
        ],
        out_specs=[
            pl.BlockSpec((R1, H), lambda i: (i, 0)),
            pl.BlockSpec((R1, H), lambda i: (i, 0)),
        ],
        out_shape=[
            jax.ShapeDtypeStruct((NP, H), jnp.float32),
            jax.ShapeDtypeStruct((NP, H), jnp.bfloat16),
        ],
    )(agg, y, deg2d, b_row, W2)


def _final_body(agg_ref, y_ref, deg_ref, b_ref, batch_ref, wo_ref, bo_ref,
                out_ref, sums_ref, cnts_ref):
    i = pl.program_id(0)

    @pl.when(i == 0)
    def _():
        sums_ref[...] = jnp.zeros_like(sums_ref)
        cnts_ref[...] = jnp.zeros_like(cnts_ref)

    dinv = _dinv_col(deg_ref)
    agg = agg_ref[0].astype(jnp.float32) + agg_ref[1].astype(jnp.float32)
    h = jnp.maximum((agg + y_ref[...]) * dinv + b_ref[...], 0.0)
    seg = _col(batch_ref[...].astype(jnp.float32))
    gids = lax.broadcasted_iota(jnp.int32, (1, G), 1).astype(jnp.float32)
    oh = (seg == gids).astype(jnp.float32)
    sums_ref[...] += lax.dot_general(
        oh, h, (((0,), (0,)), ((), ())), preferred_element_type=jnp.float32)
    cnts_ref[...] += lax.dot_general(
        oh, jnp.ones((R1, 1), jnp.float32), (((0,), (0,)), ((), ())),
        preferred_element_type=jnp.float32)

    @pl.when(i == GP - 1)
    def _():
        pooled = sums_ref[...] / jnp.maximum(cnts_ref[...], 1.0)
        out_ref[...] = jnp.dot(
            pooled, wo_ref[...], preferred_element_type=jnp.float32
        ) + bo_ref[...]


def _final(agg, y, deg2d, b_row, batch2d, Wo, bo_row):
    return pl.pallas_call(
        _final_body,
        grid=(GP,),
        in_specs=[
            pl.BlockSpec((NC, R1, H), lambda i: (0, i, 0)),
            pl.BlockSpec((R1, H), lambda i: (i, 0)),
            _deg_spec,
            pl.BlockSpec((1, H), lambda i: (0, 0)),
            pl.BlockSpec((RW, 128), lambda i: (i, 0)),
            pl.BlockSpec((H, OUT), lambda i: (0, 0)),
            pl.BlockSpec((1, OUT), lambda i: (0, 0)),
        ],
        out_specs=pl.BlockSpec((G, OUT), lambda i: (0, 0)),
        out_shape=jax.ShapeDtypeStruct((G, OUT), jnp.float32),
        scratch_shapes=[
            pltpu.VMEM((G, H), jnp.float32),
            pltpu.VMEM((G, 1), jnp.float32),
        ],
    )(agg, y, deg2d, b_row, batch2d, Wo, bo_row)


def kernel(x, edge_index, batch, W1, b1, W2, b2, Wo, bo):
    xp = jnp.pad(x, ((0, NP - N), (0, 0)))
    padrows = (N + (jnp.arange(EP - E, dtype=jnp.int32) % (NP - N))
               ).astype(jnp.int32)
    srcp = jnp.concatenate([edge_index[0], padrows]).reshape(NW, CH, K)
    dstp = jnp.concatenate([edge_index[1], padrows]).reshape(NW, CH, K)
    batch2d = jnp.concatenate(
        [batch, jnp.full((NP - N,), G, jnp.int32)]).reshape(NP // 128, 128)

    xw1 = _mm(xp, W1)
    deg_part = _deg_call(dstp)
    deg2d = deg_part.reshape(NC, NP // 128, 128)
    y1, y1h = _scale(xw1, deg2d)
    agg1 = _agg_call(srcp, dstp, y1h)
    y2, y2h = _layer(agg1, y1, deg2d, b1.reshape(1, H), W2)
    agg2 = _agg_call(srcp, dstp, y2h)
    return _final(agg2, y2, deg2d, b2.reshape(1, H), batch2d,
                  Wo, bo.reshape(1, OUT))


# speedup vs baseline: 1.0920x; 1.0065x over previous
import functools

import jax
import jax.numpy as jnp
from jax import lax
from jax.experimental import pallas as pl
from jax.experimental.pallas import tpu as pltpu
from jax.experimental.pallas import tpu_sc as plsc

N = 10000
E = 320000
D = 128
H = 64
G = 16
OUT = 64

NC = 2
NS = 16
NW = NC * NS
L = 16

K = 512
CH = 20
EW = K * CH
EP = EW * NW
NP = 10240
RPT = NP // NS
R1 = 2048
GP = NP // R1

_mesh = plsc.VectorSubcoreMesh(
    core_axis_name="c", subcore_axis_name="s", num_cores=NC, num_subcores=NS)

_sc_params = pltpu.CompilerParams(use_tc_tiling_on_sc=False)

_zeros16 = functools.partial(jnp.zeros, (L,), jnp.float32)


def _deg_body(dst_hbm, out_hbm, dst_v, ones_v, zbuf_v, deg_sh):
    cid = lax.axis_index("c")
    sid = lax.axis_index("s")
    wid = sid * NC + cid

    for t in range(K // L):
        ones_v[pl.ds(t * L, L)] = jnp.ones((L,), jnp.float32)

    def zstore(i, _):
        zbuf_v[pl.ds(i * L, L)] = _zeros16()
        return 0
    lax.fori_loop(0, RPT // L, zstore, 0)

    base = sid * RPT
    pltpu.sync_copy(zbuf_v, deg_sh.at[pl.ds(base, RPT)])
    pltpu.sync_copy(dst_hbm.at[wid], dst_v)
    plsc.subcore_barrier()

    def chunk(j, _):
        pltpu.sync_copy(ones_v, deg_sh.at[dst_v.at[j]], add=True)
        return 0
    lax.fori_loop(0, CH, chunk, 0)

    plsc.subcore_barrier()
    pltpu.sync_copy(deg_sh.at[pl.ds(base, RPT)],
                    out_hbm.at[cid, pl.ds(base, RPT)])


_deg_call = pl.kernel(
    _deg_body,
    out_type=jax.ShapeDtypeStruct((NC, NP), jnp.float32),
    mesh=_mesh,
    scratch_types=[
        pltpu.VMEM((CH, K), jnp.int32),
        pltpu.VMEM((K,), jnp.float32),
        pltpu.VMEM((RPT,), jnp.float32),
        pltpu.VMEM_SHARED((NP,), jnp.float32),
    ],
    compiler_params=_sc_params,
)


def _agg_body(src_hbm, dst_hbm, y_hbm, out_hbm, src_v, dst_v, rows_v, agg_sh,
              gs0, gs1, gs2, gs3, ss0, ss1, ss2, ss3):
    gsem = [gs0, gs1, gs2, gs3]
    ssem = [ss0, ss1, ss2, ss3]
    cid = lax.axis_index("c")
    sid = lax.axis_index("s")
    wid = sid * NC + cid

    def zstore(r, _):
        for c in range(H // (2 * L)):
            rows_v[0, r, pl.ds(c * 2 * L, 2 * L)] = jnp.zeros(
                (2 * L,), jnp.bfloat16)
        return 0
    lax.fori_loop(0, K, zstore, 0)

    base = sid * RPT
    for t in range(RPT // 128):
        pltpu.sync_copy(rows_v.at[0, pl.ds(0, 128)],
                        agg_sh.at[pl.ds(base + t * 128, 128)])
    pltpu.sync_copy(src_hbm.at[wid], src_v)
    pltpu.sync_copy(dst_hbm.at[wid], dst_v)
    plsc.subcore_barrier()

    def gather(j, buf):
        pltpu.async_copy(y_hbm.at[src_v.at[j]], rows_v.at[buf], gsem[buf])

    def gwait(buf):
        pltpu.make_async_copy(y_hbm.at[src_v.at[0]], rows_v.at[buf],
                              gsem[buf]).wait()

    def scat(j, buf):
        pltpu.async_copy(rows_v.at[buf], agg_sh.at[dst_v.at[j]], ssem[buf],
                         add=True)

    def swait(buf):
        pltpu.make_async_copy(rows_v.at[buf], agg_sh.at[pl.ds(0, K)],
                              ssem[buf]).wait()

    NB = 4
    gather(0, 0)
    gather(1, 1)

    def chunk4(jj, _):
        j = jj * NB
        for u in range(NB):
            b = u
            gwait(b)
            scat(j + u, b)
            bn = (u + 2) % NB
            if u >= 2:
                swait(bn)
            else:
                @pl.when(jj > 0)
                def _():
                    swait(bn)
            gather((j + u + 2) % CH, bn)
        return 0
    lax.fori_loop(0, CH // NB, chunk4, 0)

    gwait(0)
    gwait(1)
    swait(2)
    swait(3)

    plsc.subcore_barrier()
    pltpu.sync_copy(agg_sh.at[pl.ds(base, RPT)],
                    out_hbm.at[cid, pl.ds(base, RPT)])


_agg_call = pl.kernel(
    _agg_body,
    out_type=jax.ShapeDtypeStruct((NC, NP, H), jnp.bfloat16),
    mesh=_mesh,
    scratch_types=[
        pltpu.VMEM((CH, K), jnp.int32),
        pltpu.VMEM((CH, K), jnp.int32),
        pltpu.VMEM((4, K, H), jnp.bfloat16),
        pltpu.VMEM_SHARED((NP, H), jnp.bfloat16),
    ] + [pltpu.SemaphoreType.DMA] * 8,
    compiler_params=_sc_params,
)


RW = R1 // 128

_deg_spec = pl.BlockSpec((NC, RW, 128), lambda i: (0, i, 0))


def _col(mat):
    ident = (lax.broadcasted_iota(jnp.int32, (128, 128), 0) ==
             lax.broadcasted_iota(jnp.int32, (128, 128), 1)
             ).astype(jnp.float32)
    mt = lax.dot_general(ident, mat, (((1,), (1,)), ((), ())),
                         preferred_element_type=jnp.float32)
    return jnp.concatenate([mt[:, a:a + 1] for a in range(RW)], axis=0)


def _dinv_col(deg_ref):
    return _col(lax.rsqrt(deg_ref[0] + deg_ref[1] + 1.0))


def _mm_body(x_ref, w_ref, out_ref):
    out_ref[...] = jnp.dot(x_ref[...], w_ref[...],
                           preferred_element_type=jnp.float32)


def _mm(xp, W1):
    return pl.pallas_call(
        _mm_body,
        grid=(GP,),
        in_specs=[
            pl.BlockSpec((R1, D), lambda i: (i, 0)),
            pl.BlockSpec((D, H), lambda i: (0, 0)),
        ],
        out_specs=pl.BlockSpec((R1, H), lambda i: (i, 0)),
        out_shape=jax.ShapeDtypeStruct((NP, H), jnp.float32),
    )(xp, W1)


def _scale_body(xw_ref, deg_ref, out_ref, outh_ref):
    dinv = _dinv_col(deg_ref)
    y = xw_ref[...] * dinv
    out_ref[...] = y
    outh_ref[...] = y.astype(jnp.bfloat16)


def _scale(xw, deg2d):
    return pl.pallas_call(
        _scale_body,
        grid=(GP,),
        in_specs=[
            pl.BlockSpec((R1, H), lambda i: (i, 0)),
            _deg_spec,
        ],
        out_specs=[
            pl.BlockSpec((R1, H), lambda i: (i, 0)),
            pl.BlockSpec((R1, H), lambda i: (i, 0)),
        ],
        out_shape=[
            jax.ShapeDtypeStruct((NP, H), jnp.float32),
            jax.ShapeDtypeStruct((NP, H), jnp.bfloat16),
        ],
    )(xw, deg2d)


def _layer_body(agg_ref, y_ref, deg_ref, b_ref, w_ref, out_ref, outh_ref):
    dinv = _dinv_col(deg_ref)
    agg = agg_ref[0].astype(jnp.float32) + agg_ref[1].astype(jnp.float32)
    h = jnp.maximum((agg + y_ref[...]) * dinv + b_ref[...], 0.0)
    y2 = jnp.dot(h, w_ref[...], preferred_element_type=jnp.float32) * dinv
    out_ref[...] = y2
    outh_ref[...] = y2.astype(jnp.bfloat16)


def _layer(agg, y, deg2d, b_row, W2):
    return pl.pallas_call(
        _layer_body,
        grid=(GP,),
        in_specs=[
            pl.BlockSpec((NC, R1, H), lambda i: (0, i, 0)),
            pl.BlockSpec((R1, H), lambda i: (i, 0)),
            _deg_spec,
            pl.BlockSpec((1, H), lambda i: (0, 0)),
            pl.BlockSpec((H, H), lambda i: (0, 0)),
        ],
        out_specs=[
            pl.BlockSpec((R1, H), lambda i: (i, 0)),
            pl.BlockSpec((R1, H), lambda i: (i, 0)),
        ],
        out_shape=[
            jax.ShapeDtypeStruct((NP, H), jnp.float32),
            jax.ShapeDtypeStruct((NP, H), jnp.bfloat16),
        ],
    )(agg, y, deg2d, b_row, W2)


def _final_body(agg_ref, y_ref, deg_ref, b_ref, batch_ref, wo_ref, bo_ref,
                out_ref, sums_ref, cnts_ref):
    i = pl.program_id(0)

    @pl.when(i == 0)
    def _():
        sums_ref[...] = jnp.zeros_like(sums_ref)
        cnts_ref[...] = jnp.zeros_like(cnts_ref)

    dinv = _dinv_col(deg_ref)
    agg = agg_ref[0].astype(jnp.float32) + agg_ref[1].astype(jnp.float32)
    h = jnp.maximum((agg + y_ref[...]) * dinv + b_ref[...], 0.0)
    seg = _col(batch_ref[...].astype(jnp.float32))
    gids = lax.broadcasted_iota(jnp.int32, (1, G), 1).astype(jnp.float32)
    oh = (seg == gids).astype(jnp.float32)
    sums_ref[...] += lax.dot_general(
        oh, h, (((0,), (0,)), ((), ())), preferred_element_type=jnp.float32)
    cnts_ref[...] += lax.dot_general(
        oh, jnp.ones((R1, 1), jnp.float32), (((0,), (0,)), ((), ())),
        preferred_element_type=jnp.float32)

    @pl.when(i == GP - 1)
    def _():
        pooled = sums_ref[...] / jnp.maximum(cnts_ref[...], 1.0)
        out_ref[...] = jnp.dot(
            pooled, wo_ref[...], preferred_element_type=jnp.float32
        ) + bo_ref[...]


def _final(agg, y, deg2d, b_row, batch2d, Wo, bo_row):
    return pl.pallas_call(
        _final_body,
        grid=(GP,),
        in_specs=[
            pl.BlockSpec((NC, R1, H), lambda i: (0, i, 0)),
            pl.BlockSpec((R1, H), lambda i: (i, 0)),
            _deg_spec,
            pl.BlockSpec((1, H), lambda i: (0, 0)),
            pl.BlockSpec((RW, 128), lambda i: (i, 0)),
            pl.BlockSpec((H, OUT), lambda i: (0, 0)),
            pl.BlockSpec((1, OUT), lambda i: (0, 0)),
        ],
        out_specs=pl.BlockSpec((G, OUT), lambda i: (0, 0)),
        out_shape=jax.ShapeDtypeStruct((G, OUT), jnp.float32),
        scratch_shapes=[
            pltpu.VMEM((G, H), jnp.float32),
            pltpu.VMEM((G, 1), jnp.float32),
        ],
    )(agg, y, deg2d, b_row, batch2d, Wo, bo_row)


def kernel(x, edge_index, batch, W1, b1, W2, b2, Wo, bo):
    xp = jnp.pad(x, ((0, NP - N), (0, 0)))
    padrows = (N + (jnp.arange(EP - E, dtype=jnp.int32) % (NP - N))
               ).astype(jnp.int32)
    srcp = jnp.concatenate([edge_index[0], padrows]).reshape(NW, CH, K)
    dstp = jnp.concatenate([edge_index[1], padrows]).reshape(NW, CH, K)
    batch2d = jnp.concatenate(
        [batch, jnp.full((NP - N,), G, jnp.int32)]).reshape(NP // 128, 128)

    xw1 = _mm(xp, W1)
    deg_part = _deg_call(dstp)
    deg2d = deg_part.reshape(NC, NP // 128, 128)
    y1, y1h = _scale(xw1, deg2d)
    agg1 = _agg_call(srcp, dstp, y1h)
    y2, y2h = _layer(agg1, y1, deg2d, b1.reshape(1, H), W2)
    agg2 = _agg_call(srcp, dstp, y2h)
    return _final(agg2, y2, deg2d, b2.reshape(1, H), batch2d,
                  Wo, bo.reshape(1, OUT))


# speedup vs baseline: 1.1461x; 1.0495x over previous
import functools

import jax
import jax.numpy as jnp
from jax import lax
from jax.experimental import pallas as pl
from jax.experimental.pallas import tpu as pltpu
from jax.experimental.pallas import tpu_sc as plsc

N = 10000
E = 320000
D = 128
H = 64
G = 16
OUT = 64

NC = 2
NS = 16
NW = NC * NS
L = 16

K = 500
CH = 20
EW = K * CH
EP = EW * NW
NP = 10240
RPT = NP // NS
R1 = 2048
GP = NP // R1

_mesh = plsc.VectorSubcoreMesh(
    core_axis_name="c", subcore_axis_name="s", num_cores=NC, num_subcores=NS)

_sc_params = pltpu.CompilerParams(use_tc_tiling_on_sc=False)

_zeros16 = functools.partial(jnp.zeros, (L,), jnp.float32)


def _deg_body(dst_hbm, out_hbm, dst_v, ones_v, zbuf_v, deg_sh):
    cid = lax.axis_index("c")
    sid = lax.axis_index("s")
    wid = sid * NC + cid

    for t in range(512 // L):
        ones_v[pl.ds(t * L, L)] = jnp.ones((L,), jnp.float32)

    def zstore(i, _):
        zbuf_v[pl.ds(i * L, L)] = _zeros16()
        return 0
    lax.fori_loop(0, RPT // L, zstore, 0)

    base = sid * RPT
    pltpu.sync_copy(zbuf_v, deg_sh.at[pl.ds(base, RPT)])
    pltpu.sync_copy(dst_hbm.at[wid], dst_v)
    plsc.subcore_barrier()

    def chunk(j, _):
        pltpu.sync_copy(ones_v.at[pl.ds(0, K)], deg_sh.at[dst_v.at[j]],
                        add=True)
        return 0
    lax.fori_loop(0, CH, chunk, 0)

    plsc.subcore_barrier()
    pltpu.sync_copy(deg_sh.at[pl.ds(base, RPT)],
                    out_hbm.at[cid, pl.ds(base, RPT)])


_deg_call = pl.kernel(
    _deg_body,
    out_type=jax.ShapeDtypeStruct((NC, NP), jnp.float32),
    mesh=_mesh,
    scratch_types=[
        pltpu.VMEM((CH, K), jnp.int32),
        pltpu.VMEM((512,), jnp.float32),
        pltpu.VMEM((RPT,), jnp.float32),
        pltpu.VMEM_SHARED((NP,), jnp.float32),
    ],
    compiler_params=_sc_params,
)


def _agg_body(src_hbm, dst_hbm, y_hbm, out_hbm, src_v, dst_v, rows_v, agg_sh,
              gs0, gs1, gs2, gs3, ss0, ss1, ss2, ss3):
    gsem = [gs0, gs1, gs2, gs3]
    ssem = [ss0, ss1, ss2, ss3]
    cid = lax.axis_index("c")
    sid = lax.axis_index("s")
    wid = sid * NC + cid

    def zstore(r, _):
        for c in range(H // (2 * L)):
            rows_v[0, r, pl.ds(c * 2 * L, 2 * L)] = jnp.zeros(
                (2 * L,), jnp.bfloat16)
        return 0
    lax.fori_loop(0, 128, zstore, 0)

    base = sid * RPT
    for t in range(RPT // 128):
        pltpu.sync_copy(rows_v.at[0, pl.ds(0, 128)],
                        agg_sh.at[pl.ds(base + t * 128, 128)])
    pltpu.sync_copy(src_hbm.at[wid], src_v)
    pltpu.sync_copy(dst_hbm.at[wid], dst_v)
    plsc.subcore_barrier()

    def gather(j, buf):
        pltpu.async_copy(y_hbm.at[src_v.at[j]], rows_v.at[buf], gsem[buf])

    def gwait(buf):
        pltpu.make_async_copy(y_hbm.at[src_v.at[0]], rows_v.at[buf],
                              gsem[buf]).wait()

    def scat(j, buf):
        pltpu.async_copy(rows_v.at[buf], agg_sh.at[dst_v.at[j]], ssem[buf],
                         add=True)

    def swait(buf):
        pltpu.make_async_copy(rows_v.at[buf], agg_sh.at[pl.ds(0, K)],
                              ssem[buf]).wait()

    NB = 4
    gather(0, 0)
    gather(1, 1)

    def chunk4(jj, _):
        j = jj * NB
        for u in range(NB):
            b = u
            gwait(b)
            scat(j + u, b)
            bn = (u + 2) % NB
            if u >= 2:
                swait(bn)
            else:
                @pl.when(jj > 0)
                def _():
                    swait(bn)
            gather((j + u + 2) % CH, bn)
        return 0
    lax.fori_loop(0, CH // NB, chunk4, 0)

    gwait(0)
    gwait(1)
    swait(2)
    swait(3)

    plsc.subcore_barrier()
    pltpu.sync_copy(agg_sh.at[pl.ds(base, RPT)],
                    out_hbm.at[cid, pl.ds(base, RPT)])


_agg_call = pl.kernel(
    _agg_body,
    out_type=jax.ShapeDtypeStruct((NC, NP, H), jnp.bfloat16),
    mesh=_mesh,
    scratch_types=[
        pltpu.VMEM((CH, K), jnp.int32),
        pltpu.VMEM((CH, K), jnp.int32),
        pltpu.VMEM((4, K, H), jnp.bfloat16),
        pltpu.VMEM_SHARED((NP, H), jnp.bfloat16),
    ] + [pltpu.SemaphoreType.DMA] * 8,
    compiler_params=_sc_params,
)


RW = R1 // 128

_deg_spec = pl.BlockSpec((NC, RW, 128), lambda i: (0, i, 0))


def _col(mat):
    ident = (lax.broadcasted_iota(jnp.int32, (128, 128), 0) ==
             lax.broadcasted_iota(jnp.int32, (128, 128), 1)
             ).astype(jnp.float32)
    mt = lax.dot_general(ident, mat, (((1,), (1,)), ((), ())),
                         preferred_element_type=jnp.float32)
    return jnp.concatenate([mt[:, a:a + 1] for a in range(RW)], axis=0)


def _dinv_col(deg_ref):
    return _col(lax.rsqrt(deg_ref[0] + deg_ref[1] + 1.0))


def _mm_scale_body(x_ref, w_ref, deg_ref, outh_ref):
    dinv = _dinv_col(deg_ref)
    y = jnp.dot(x_ref[...], w_ref[...],
                preferred_element_type=jnp.float32) * dinv
    outh_ref[...] = y.astype(jnp.bfloat16)


def _mm_scale(xp, W1, deg2d):
    return pl.pallas_call(
        _mm_scale_body,
        grid=(GP,),
        in_specs=[
            pl.BlockSpec((R1, D), lambda i: (i, 0)),
            pl.BlockSpec((D, H), lambda i: (0, 0)),
            _deg_spec,
        ],
        out_specs=pl.BlockSpec((R1, H), lambda i: (i, 0)),
        out_shape=jax.ShapeDtypeStruct((NP, H), jnp.bfloat16),
    )(xp, W1, deg2d)


def _layer_body(agg_ref, y_ref, deg_ref, b_ref, w_ref, outh_ref):
    dinv = _dinv_col(deg_ref)
    agg = (agg_ref[0].astype(jnp.float32) + agg_ref[1].astype(jnp.float32)
           + y_ref[...].astype(jnp.float32))
    h = jnp.maximum(agg * dinv + b_ref[...], 0.0)
    y2 = jnp.dot(h, w_ref[...], preferred_element_type=jnp.float32) * dinv
    outh_ref[...] = y2.astype(jnp.bfloat16)


def _layer(agg, yh, deg2d, b_row, W2):
    return pl.pallas_call(
        _layer_body,
        grid=(GP,),
        in_specs=[
            pl.BlockSpec((NC, R1, H), lambda i: (0, i, 0)),
            pl.BlockSpec((R1, H), lambda i: (i, 0)),
            _deg_spec,
            pl.BlockSpec((1, H), lambda i: (0, 0)),
            pl.BlockSpec((H, H), lambda i: (0, 0)),
        ],
        out_specs=pl.BlockSpec((R1, H), lambda i: (i, 0)),
        out_shape=jax.ShapeDtypeStruct((NP, H), jnp.bfloat16),
    )(agg, yh, deg2d, b_row, W2)


def _final_body(agg_ref, y_ref, deg_ref, b_ref, batch_ref, wo_ref, bo_ref,
                out_ref, sums_ref, cnts_ref):
    i = pl.program_id(0)

    @pl.when(i == 0)
    def _():
        sums_ref[...] = jnp.zeros_like(sums_ref)
        cnts_ref[...] = jnp.zeros_like(cnts_ref)

    dinv = _dinv_col(deg_ref)
    agg = (agg_ref[0].astype(jnp.float32) + agg_ref[1].astype(jnp.float32)
           + y_ref[...].astype(jnp.float32))
    h = jnp.maximum(agg * dinv + b_ref[...], 0.0)
    seg = _col(batch_ref[...].astype(jnp.float32))
    gids = lax.broadcasted_iota(jnp.int32, (1, G), 1).astype(jnp.float32)
    oh = (seg == gids).astype(jnp.float32)
    sums_ref[...] += lax.dot_general(
        oh, h, (((0,), (0,)), ((), ())), preferred_element_type=jnp.float32)
    cnts_ref[...] += lax.dot_general(
        oh, jnp.ones((R1, 1), jnp.float32), (((0,), (0,)), ((), ())),
        preferred_element_type=jnp.float32)

    @pl.when(i == GP - 1)
    def _():
        pooled = sums_ref[...] / jnp.maximum(cnts_ref[...], 1.0)
        out_ref[...] = jnp.dot(
            pooled, wo_ref[...], preferred_element_type=jnp.float32
        ) + bo_ref[...]


def _final(agg, y, deg2d, b_row, batch2d, Wo, bo_row):
    return pl.pallas_call(
        _final_body,
        grid=(GP,),
        in_specs=[
            pl.BlockSpec((NC, R1, H), lambda i: (0, i, 0)),
            pl.BlockSpec((R1, H), lambda i: (i, 0)),
            _deg_spec,
            pl.BlockSpec((1, H), lambda i: (0, 0)),
            pl.BlockSpec((RW, 128), lambda i: (i, 0)),
            pl.BlockSpec((H, OUT), lambda i: (0, 0)),
            pl.BlockSpec((1, OUT), lambda i: (0, 0)),
        ],
        out_specs=pl.BlockSpec((G, OUT), lambda i: (0, 0)),
        out_shape=jax.ShapeDtypeStruct((G, OUT), jnp.float32),
        scratch_shapes=[
            pltpu.VMEM((G, H), jnp.float32),
            pltpu.VMEM((G, 1), jnp.float32),
        ],
    )(agg, y, deg2d, b_row, batch2d, Wo, bo_row)


def kernel(x, edge_index, batch, W1, b1, W2, b2, Wo, bo):
    xp = jnp.pad(x, ((0, NP - N), (0, 0)))
    srcp = edge_index[0].reshape(NW, CH, K)
    dstp = edge_index[1].reshape(NW, CH, K)
    batch2d = jnp.concatenate(
        [batch, jnp.full((NP - N,), G, jnp.int32)]).reshape(NP // 128, 128)

    deg_part = _deg_call(dstp)
    deg2d = deg_part.reshape(NC, NP // 128, 128)
    y1h = _mm_scale(xp, W1, deg2d)
    agg1 = _agg_call(srcp, dstp, y1h)
    y2h = _layer(agg1, y1h, deg2d, b1.reshape(1, H), W2)
    agg2 = _agg_call(srcp, dstp, y2h)
    return _final(agg2, y2h, deg2d, b2.reshape(1, H), batch2d,
                  Wo, bo.reshape(1, OUT))


# speedup vs baseline: 1.1699x; 1.0208x over previous
import functools

import jax
import jax.numpy as jnp
from jax import lax
from jax.experimental import pallas as pl
from jax.experimental.pallas import tpu as pltpu
from jax.experimental.pallas import tpu_sc as plsc

N = 10000
E = 320000
D = 128
H = 64
G = 16
OUT = 64

NC = 2
NS = 16
NW = NC * NS
L = 16

K = 500
CH = 20
EW = K * CH
EP = EW * NW
NP = 10240
RPT = NP // NS
R1 = 5120
GP = NP // R1

_mesh = plsc.VectorSubcoreMesh(
    core_axis_name="c", subcore_axis_name="s", num_cores=NC, num_subcores=NS)

_sc_params = pltpu.CompilerParams(use_tc_tiling_on_sc=False)

_zeros16 = functools.partial(jnp.zeros, (L,), jnp.float32)


def _deg_body(dst_hbm, out_hbm, dst_v, ones_v, zbuf_v, deg_sh):
    cid = lax.axis_index("c")
    sid = lax.axis_index("s")
    wid = sid * NC + cid

    for t in range(512 // L):
        ones_v[pl.ds(t * L, L)] = jnp.ones((L,), jnp.float32)

    def zstore(i, _):
        zbuf_v[pl.ds(i * L, L)] = _zeros16()
        return 0
    lax.fori_loop(0, RPT // L, zstore, 0)

    base = sid * RPT
    pltpu.sync_copy(zbuf_v, deg_sh.at[pl.ds(base, RPT)])
    pltpu.sync_copy(dst_hbm.at[wid], dst_v)
    plsc.subcore_barrier()

    def chunk(j, _):
        pltpu.sync_copy(ones_v.at[pl.ds(0, K)], deg_sh.at[dst_v.at[j]],
                        add=True)
        return 0
    lax.fori_loop(0, CH, chunk, 0)

    plsc.subcore_barrier()
    pltpu.sync_copy(deg_sh.at[pl.ds(base, RPT)],
                    out_hbm.at[cid, pl.ds(base, RPT)])


_deg_call = pl.kernel(
    _deg_body,
    out_type=jax.ShapeDtypeStruct((NC, NP), jnp.float32),
    mesh=_mesh,
    scratch_types=[
        pltpu.VMEM((CH, K), jnp.int32),
        pltpu.VMEM((512,), jnp.float32),
        pltpu.VMEM((RPT,), jnp.float32),
        pltpu.VMEM_SHARED((NP,), jnp.float32),
    ],
    compiler_params=_sc_params,
)


def _agg_body(src_hbm, dst_hbm, y_hbm, out_hbm, src_v, dst_v, rows_v, agg_sh,
              gs0, gs1, gs2, gs3, ss0, ss1, ss2, ss3):
    gsem = [gs0, gs1, gs2, gs3]
    ssem = [ss0, ss1, ss2, ss3]
    cid = lax.axis_index("c")
    sid = lax.axis_index("s")
    wid = sid * NC + cid

    def zstore(r, _):
        for c in range(H // (2 * L)):
            rows_v[0, r, pl.ds(c * 2 * L, 2 * L)] = jnp.zeros(
                (2 * L,), jnp.bfloat16)
        return 0
    lax.fori_loop(0, 128, zstore, 0)

    base = sid * RPT
    for t in range(RPT // 128):
        pltpu.sync_copy(rows_v.at[0, pl.ds(0, 128)],
                        agg_sh.at[pl.ds(base + t * 128, 128)])
    pltpu.sync_copy(src_hbm.at[wid], src_v)
    pltpu.sync_copy(dst_hbm.at[wid], dst_v)
    plsc.subcore_barrier()

    def gather(j, buf):
        pltpu.async_copy(y_hbm.at[src_v.at[j]], rows_v.at[buf], gsem[buf])

    def gwait(buf):
        pltpu.make_async_copy(y_hbm.at[src_v.at[0]], rows_v.at[buf],
                              gsem[buf]).wait()

    def scat(j, buf):
        pltpu.async_copy(rows_v.at[buf], agg_sh.at[dst_v.at[j]], ssem[buf],
                         add=True)

    def swait(buf):
        pltpu.make_async_copy(rows_v.at[buf], agg_sh.at[pl.ds(0, K)],
                              ssem[buf]).wait()

    NB = 4
    gather(0, 0)
    gather(1, 1)

    def chunk4(jj, _):
        j = jj * NB
        for u in range(NB):
            b = u
            gwait(b)
            scat(j + u, b)
            bn = (u + 2) % NB
            if u >= 2:
                swait(bn)
            else:
                @pl.when(jj > 0)
                def _():
                    swait(bn)
            gather((j + u + 2) % CH, bn)
        return 0
    lax.fori_loop(0, CH // NB, chunk4, 0)

    gwait(0)
    gwait(1)
    swait(2)
    swait(3)

    plsc.subcore_barrier()
    pltpu.sync_copy(agg_sh.at[pl.ds(base, RPT)],
                    out_hbm.at[cid, pl.ds(base, RPT)])


_agg_call = pl.kernel(
    _agg_body,
    out_type=jax.ShapeDtypeStruct((NC, NP, H), jnp.bfloat16),
    mesh=_mesh,
    scratch_types=[
        pltpu.VMEM((CH, K), jnp.int32),
        pltpu.VMEM((CH, K), jnp.int32),
        pltpu.VMEM((4, K, H), jnp.bfloat16),
        pltpu.VMEM_SHARED((NP, H), jnp.bfloat16),
    ] + [pltpu.SemaphoreType.DMA] * 8,
    compiler_params=_sc_params,
)


RW = R1 // 128

_deg_spec = pl.BlockSpec((NC, RW, 128), lambda i: (0, i, 0))


def _col(mat):
    ident = (lax.broadcasted_iota(jnp.int32, (128, 128), 0) ==
             lax.broadcasted_iota(jnp.int32, (128, 128), 1)
             ).astype(jnp.float32)
    mt = lax.dot_general(ident, mat, (((1,), (1,)), ((), ())),
                         preferred_element_type=jnp.float32)
    return jnp.concatenate([mt[:, a:a + 1] for a in range(RW)], axis=0)


def _dinv_col(deg_ref):
    return _col(lax.rsqrt(deg_ref[0] + deg_ref[1] + 1.0))


def _mm_scale_body(x_ref, w_ref, deg_ref, outh_ref):
    dinv = _dinv_col(deg_ref)
    y = jnp.dot(x_ref[...], w_ref[...],
                preferred_element_type=jnp.float32) * dinv
    outh_ref[...] = y.astype(jnp.bfloat16)


def _mm_scale(xp, W1, deg2d):
    return pl.pallas_call(
        _mm_scale_body,
        grid=(GP,),
        in_specs=[
            pl.BlockSpec((R1, D), lambda i: (i, 0)),
            pl.BlockSpec((D, H), lambda i: (0, 0)),
            _deg_spec,
        ],
        out_specs=pl.BlockSpec((R1, H), lambda i: (i, 0)),
        out_shape=jax.ShapeDtypeStruct((NP, H), jnp.bfloat16),
    )(xp, W1, deg2d)


def _layer_body(agg_ref, y_ref, deg_ref, b_ref, w_ref, outh_ref):
    dinv = _dinv_col(deg_ref)
    agg = (agg_ref[0].astype(jnp.float32) + agg_ref[1].astype(jnp.float32)
           + y_ref[...].astype(jnp.float32))
    h = jnp.maximum(agg * dinv + b_ref[...], 0.0)
    y2 = jnp.dot(h, w_ref[...], preferred_element_type=jnp.float32) * dinv
    outh_ref[...] = y2.astype(jnp.bfloat16)


def _layer(agg, yh, deg2d, b_row, W2):
    return pl.pallas_call(
        _layer_body,
        grid=(GP,),
        in_specs=[
            pl.BlockSpec((NC, R1, H), lambda i: (0, i, 0)),
            pl.BlockSpec((R1, H), lambda i: (i, 0)),
            _deg_spec,
            pl.BlockSpec((1, H), lambda i: (0, 0)),
            pl.BlockSpec((H, H), lambda i: (0, 0)),
        ],
        out_specs=pl.BlockSpec((R1, H), lambda i: (i, 0)),
        out_shape=jax.ShapeDtypeStruct((NP, H), jnp.bfloat16),
    )(agg, yh, deg2d, b_row, W2)


def _final_body(agg_ref, y_ref, deg_ref, b_ref, batch_ref, wo_ref, bo_ref,
                out_ref, sums_ref, cnts_ref):
    i = pl.program_id(0)

    @pl.when(i == 0)
    def _():
        sums_ref[...] = jnp.zeros_like(sums_ref)
        cnts_ref[...] = jnp.zeros_like(cnts_ref)

    dinv = _dinv_col(deg_ref)
    agg = (agg_ref[0].astype(jnp.float32) + agg_ref[1].astype(jnp.float32)
           + y_ref[...].astype(jnp.float32))
    h = jnp.maximum(agg * dinv + b_ref[...], 0.0)
    seg = _col(batch_ref[...].astype(jnp.float32))
    gids = lax.broadcasted_iota(jnp.int32, (1, G), 1).astype(jnp.float32)
    oh = (seg == gids).astype(jnp.float32)
    sums_ref[...] += lax.dot_general(
        oh, h, (((0,), (0,)), ((), ())), preferred_element_type=jnp.float32)
    cnts_ref[...] += lax.dot_general(
        oh, jnp.ones((R1, 1), jnp.float32), (((0,), (0,)), ((), ())),
        preferred_element_type=jnp.float32)

    @pl.when(i == GP - 1)
    def _():
        pooled = sums_ref[...] / jnp.maximum(cnts_ref[...], 1.0)
        out_ref[...] = jnp.dot(
            pooled, wo_ref[...], preferred_element_type=jnp.float32
        ) + bo_ref[...]


def _final(agg, y, deg2d, b_row, batch2d, Wo, bo_row):
    return pl.pallas_call(
        _final_body,
        grid=(GP,),
        in_specs=[
            pl.BlockSpec((NC, R1, H), lambda i: (0, i, 0)),
            pl.BlockSpec((R1, H), lambda i: (i, 0)),
            _deg_spec,
            pl.BlockSpec((1, H), lambda i: (0, 0)),
            pl.BlockSpec((RW, 128), lambda i: (i, 0)),
            pl.BlockSpec((H, OUT), lambda i: (0, 0)),
            pl.BlockSpec((1, OUT), lambda i: (0, 0)),
        ],
        out_specs=pl.BlockSpec((G, OUT), lambda i: (0, 0)),
        out_shape=jax.ShapeDtypeStruct((G, OUT), jnp.float32),
        scratch_shapes=[
            pltpu.VMEM((G, H), jnp.float32),
            pltpu.VMEM((G, 1), jnp.float32),
        ],
    )(agg, y, deg2d, b_row, batch2d, Wo, bo_row)


def kernel(x, edge_index, batch, W1, b1, W2, b2, Wo, bo):
    xp = jnp.pad(x, ((0, NP - N), (0, 0)))
    srcp = edge_index[0].reshape(NW, CH, K)
    dstp = edge_index[1].reshape(NW, CH, K)
    batch2d = jnp.concatenate(
        [batch, jnp.full((NP - N,), G, jnp.int32)]).reshape(NP // 128, 128)

    deg_part = _deg_call(dstp)
    deg2d = deg_part.reshape(NC, NP // 128, 128)
    y1h = _mm_scale(xp, W1, deg2d)
    agg1 = _agg_call(srcp, dstp, y1h)
    y2h = _layer(agg1, y1h, deg2d, b1.reshape(1, H), W2)
    agg2 = _agg_call(srcp, dstp, y2h)
    return _final(agg2, y2h, deg2d, b2.reshape(1, H), batch2d,
                  Wo, bo.reshape(1, OUT))
